# Initial kernel scaffold; baseline (speedup 1.0000x reference)
import jax
import jax.numpy as jnp
from jax.experimental import pallas as pl


def kernel(z, codebook):
    raise NotImplementedError("write your pallas kernel here")


# speedup vs baseline: 2.3999x; 2.3999x over previous
import jax
import jax.numpy as jnp
from jax.experimental import pallas as pl
from jax.experimental.pallas import tpu as pltpu

_K = 8192
_D = 32
_TB = 256


def _vq_body(z_ref, cbt_ref, cb_ref, out_ref, loss_ref, acc_ref):
    i = pl.program_id(0)
    zb = z_ref[...]
    cbt = cbt_ref[...]

    zsq = jnp.sum(zb * zb, axis=1, keepdims=True)
    csq = jnp.sum(cbt * cbt, axis=0, keepdims=True)
    mm = jnp.dot(zb, cbt)
    d = zsq - 2.0 * mm + csq

    min_d = jnp.min(d, axis=1, keepdims=True)
    lane = jax.lax.broadcasted_iota(jnp.int32, (_TB, _K), 1)
    idx = jnp.min(
        jnp.where(d == min_d, lane, jnp.int32(2**30)), axis=1
    )
    onehot = (lane == idx[:, None]).astype(jnp.float32)
    zq = jnp.dot(onehot, cb_ref[...])

    out_ref[...] = zb + (zq - zb)

    part = jnp.sum((zq - zb) ** 2)

    @pl.when(i == 0)
    def _():
        acc_ref[0, 0] = 0.0

    acc_ref[0, 0] += part

    @pl.when(i == pl.num_programs(0) - 1)
    def _():
        m = acc_ref[0, 0] / (8192.0 * 32.0)
        loss_ref[...] = jnp.full((1, 1), m + 0.25 * m, jnp.float32)


def kernel(z, codebook):
    B, C, H, W = z.shape
    z_flat = jnp.transpose(z, (0, 2, 3, 1)).reshape(-1, _D)
    cbt = codebook.T

    zq_flat, loss = pl.pallas_call(
        _vq_body,
        grid=(z_flat.shape[0] // _TB,),
        in_specs=[
            pl.BlockSpec((_TB, _D), lambda i: (i, 0)),
            pl.BlockSpec((_D, _K), lambda i: (0, 0)),
            pl.BlockSpec((_K, _D), lambda i: (0, 0)),
        ],
        out_specs=[
            pl.BlockSpec((_TB, _D), lambda i: (i, 0)),
            pl.BlockSpec((1, 1), lambda i: (0, 0)),
        ],
        out_shape=[
            jax.ShapeDtypeStruct((z_flat.shape[0], _D), jnp.float32),
            jax.ShapeDtypeStruct((1, 1), jnp.float32),
        ],
        scratch_shapes=[pltpu.SMEM((1, 1), jnp.float32)],
    )(z_flat, cbt, codebook)

    z_qst = jnp.transpose(zq_flat.reshape(B, H, W, C), (0, 3, 1, 2))
    return (z_qst, loss.reshape(()))


# speedup vs baseline: 3.8921x; 1.6218x over previous
import functools

import jax
import jax.numpy as jnp
from jax import lax
from jax.experimental import pallas as pl
from jax.experimental.pallas import tpu as pltpu
from jax.experimental.pallas import tpu_sc as plsc

_K = 8192
_D = 32
_TB = 256
_N = 8192

_NC = 2
_NS = 16
_NW = _NC * _NS
_BPW = _N // _NW


def _vq_tc_body(z_ref, cbt_ref, idx_ref, loss_ref, acc_ref):
    i = pl.program_id(0)
    zb = z_ref[...]
    cbt = cbt_ref[...]

    zsq = jnp.sum(zb * zb, axis=1, keepdims=True)
    csq = jnp.sum(cbt * cbt, axis=0, keepdims=True)
    mm = jnp.dot(zb, cbt)
    d = zsq - 2.0 * mm + csq

    min_d = jnp.min(d, axis=1, keepdims=True)
    lane = jax.lax.broadcasted_iota(jnp.int32, (_TB, _K), 1)
    idx = jnp.min(
        jnp.where(d == min_d, lane, jnp.int32(2**30)), axis=1
    )
    idx_ref[...] = idx.reshape(1, 1, _TB)

    part = jnp.sum(min_d)

    @pl.when(i == 0)
    def _():
        acc_ref[0, 0] = 0.0

    acc_ref[0, 0] += part

    @pl.when(i == pl.num_programs(0) - 1)
    def _():
        m = acc_ref[0, 0] / (_N * 32.0)
        loss_ref[...] = jnp.full((1, 1), m + 0.25 * m, jnp.float32)


def _sc_gather_body(cb_hbm, idx_hbm, out_hbm, idx_v, rows_v, sem):
    wid = lax.axis_index("s") * _NC + lax.axis_index("c")
    base = wid * _BPW
    pltpu.sync_copy(idx_hbm.at[pl.ds(base, _BPW)], idx_v)
    pltpu.async_copy(cb_hbm.at[idx_v], rows_v, sem).wait()
    pltpu.sync_copy(rows_v, out_hbm.at[pl.ds(base, _BPW)])


def kernel(z, codebook):
    B, C, H, W = z.shape
    z_flat = jnp.transpose(z, (0, 2, 3, 1)).reshape(-1, _D)
    cbt = codebook.T

    idx3, loss = pl.pallas_call(
        _vq_tc_body,
        grid=(_N // _TB,),
        in_specs=[
            pl.BlockSpec((_TB, _D), lambda i: (i, 0)),
            pl.BlockSpec((_D, _K), lambda i: (0, 0)),
        ],
        out_specs=[
            pl.BlockSpec((1, 1, _TB), lambda i: (i, 0, 0)),
            pl.BlockSpec((1, 1), lambda i: (0, 0)),
        ],
        out_shape=[
            jax.ShapeDtypeStruct((_N // _TB, 1, _TB), jnp.int32),
            jax.ShapeDtypeStruct((1, 1), jnp.float32),
        ],
        scratch_shapes=[pltpu.SMEM((1, 1), jnp.float32)],
    )(z_flat, cbt)

    idx = idx3.reshape(_N)

    sc_gather = functools.partial(
        pl.kernel,
        mesh=plsc.VectorSubcoreMesh(core_axis_name="c", subcore_axis_name="s"),
        out_type=jax.ShapeDtypeStruct((_N, _D), jnp.float32),
        scratch_types=[
            pltpu.VMEM((_BPW,), jnp.int32),
            pltpu.VMEM((_BPW, _D), jnp.float32),
            pltpu.SemaphoreType.DMA,
        ],
        compiler_params=pltpu.CompilerParams(use_tc_tiling_on_sc=False),
    )(_sc_gather_body)

    zq_flat = sc_gather(codebook, idx)

    z_qst = jnp.transpose(zq_flat.reshape(B, H, W, C), (0, 3, 1, 2))
    return (z_qst, loss.reshape(()))


# speedup vs baseline: 4.8130x; 1.2366x over previous
import functools

import jax
import jax.numpy as jnp
from jax import lax
from jax.experimental import pallas as pl
from jax.experimental.pallas import tpu as pltpu
from jax.experimental.pallas import tpu_sc as plsc

_K = 8192
_D = 32
_TB = 512
_N = 8192
_HW = 1024
_LC = 128
_RB = 64

_NC = 2
_NS = 16
_NW = _NC * _NS
_BPW = _N // _NW


def _vq_tc_body(z_ref, cbt_ref, idx_ref, loss_ref, acc_ref):
    i = pl.program_id(0)
    zc = z_ref[0]
    zb = zc.T
    cbt2 = cbt_ref[...] * -2.0

    zsq = jnp.sum(zb * zb, axis=1, keepdims=True)
    csq = jnp.sum(cbt_ref[...] * cbt_ref[...], axis=0, keepdims=True)
    mm2 = jnp.dot(zb, cbt2)

    part = jnp.float32(0.0)
    lane = jax.lax.broadcasted_iota(jnp.int32, (_RB, _LC), 1)
    for r in range(_TB // _RB):
        rlo, rhi = r * _RB, (r + 1) * _RB
        zsq_r = zsq[rlo:rhi]
        minv = (zsq_r + mm2[rlo:rhi, 0:_LC]) + csq[:, 0:_LC]
        mini = jnp.zeros((_RB, _LC), jnp.int32)
        for j in range(1, _K // _LC):
            lo, hi = j * _LC, (j + 1) * _LC
            dj = (zsq_r + mm2[rlo:rhi, lo:hi]) + csq[:, lo:hi]
            upd = dj < minv
            mini = jnp.where(upd, jnp.int32(j), mini)
            minv = jnp.minimum(minv, dj)

        m = jnp.min(minv, axis=1, keepdims=True)
        kfull = mini * _LC + lane
        idx = jnp.min(
            jnp.where(minv == m, kfull, jnp.int32(2**30)), axis=1
        )
        idx_ref[rlo:rhi] = idx
        part = part + jnp.sum(m)

    @pl.when(i == 0)
    def _():
        acc_ref[0, 0] = 0.0

    acc_ref[0, 0] += part

    @pl.when(i == pl.num_programs(0) - 1)
    def _():
        mean = acc_ref[0, 0] / (_N * 32.0)
        loss_ref[...] = jnp.full((1, 1), mean + 0.25 * mean, jnp.float32)


def _sc_gather_body(cb_hbm, idx_hbm, out_hbm, idx_v, rows_v, sem):
    wid = lax.axis_index("s") * _NC + lax.axis_index("c")
    base = wid * _BPW
    pltpu.sync_copy(idx_hbm.at[pl.ds(base, _BPW)], idx_v)
    pltpu.async_copy(cb_hbm.at[idx_v], rows_v, sem).wait()
    pltpu.sync_copy(rows_v, out_hbm.at[pl.ds(base, _BPW)])


def kernel(z, codebook):
    B, C, H, W = z.shape
    z3 = z.reshape(B, C, H * W)
    cbt = codebook.T
    bpb = _HW // _TB

    idx3, loss = pl.pallas_call(
        _vq_tc_body,
        grid=(_N // _TB,),
        in_specs=[
            pl.BlockSpec((1, _D, _TB), lambda i: (i // bpb, 0, i % bpb)),
            pl.BlockSpec((_D, _K), lambda i: (0, 0)),
        ],
        out_specs=[
            pl.BlockSpec((_TB,), lambda i: (i,)),
            pl.BlockSpec((1, 1), lambda i: (0, 0)),
        ],
        out_shape=[
            jax.ShapeDtypeStruct((_N,), jnp.int32),
            jax.ShapeDtypeStruct((1, 1), jnp.float32),
        ],
        scratch_shapes=[pltpu.SMEM((1, 1), jnp.float32)],
    )(z3, cbt)

    idx = idx3

    sc_gather = functools.partial(
        pl.kernel,
        mesh=plsc.VectorSubcoreMesh(core_axis_name="c", subcore_axis_name="s"),
        out_type=jax.ShapeDtypeStruct((_N, _D), jnp.float32),
        scratch_types=[
            pltpu.VMEM((_BPW,), jnp.int32),
            pltpu.VMEM((_BPW, _D), jnp.float32),
            pltpu.SemaphoreType.DMA,
        ],
        compiler_params=pltpu.CompilerParams(use_tc_tiling_on_sc=False),
    )(_sc_gather_body)

    zq_flat = sc_gather(codebook, idx)

    z_qst = jnp.transpose(zq_flat.reshape(B, H, W, C), (0, 3, 1, 2))
    return (z_qst, loss.reshape(()))
